# baseline (speedup 1.0000x reference)
import functools

import jax
import jax.numpy as jnp
from jax import lax
from jax.experimental import pallas as pl
from jax.experimental.pallas import tpu as pltpu
from jax.experimental.pallas import tpu_sc as plsc

B, S, D = 4, 8192, 768
HIDDEN = 64
MAX_K = 64
N = B * S
CHUNK = 2048
NSTEP = N // CHUNK
NSLOT = B * MAX_K
R, C = 32, 1024
RPB = R // B


def _mlp_zero_body(x_ref, w1_ref, b1_ref, w2t_ref, b2_ref, logits_ref, o_ref):
    x = x_ref[...]
    h = lax.dot_general(x, w1_ref[...], (((1,), (0,)), ((), ())))
    h = jnp.maximum(h + b1_ref[...], 0.0)
    lg = lax.dot_general(w2t_ref[...], h, (((1,), (1,)), ((), ())))
    lg = lg + b2_ref[...]
    logits_ref[...] = lg.reshape(1, 1, CHUNK)
    o_ref[...] = jnp.zeros((CHUNK, D), jnp.float32)


def _mlp_zero(x, w1, b1r, w2t, b2r):
    return pl.pallas_call(
        _mlp_zero_body,
        grid=(NSTEP,),
        in_specs=[
            pl.BlockSpec((CHUNK, D), lambda i: (i, 0)),
            pl.BlockSpec((D, HIDDEN), lambda i: (0, 0)),
            pl.BlockSpec((1, HIDDEN), lambda i: (0, 0)),
            pl.BlockSpec((1, HIDDEN), lambda i: (0, 0)),
            pl.BlockSpec((1, 1), lambda i: (0, 0)),
        ],
        out_specs=[
            pl.BlockSpec((1, 1, CHUNK), lambda i: (i, 0, 0)),
            pl.BlockSpec((CHUNK, D), lambda i: (i, 0)),
        ],
        out_shape=[
            jax.ShapeDtypeStruct((NSTEP, 1, CHUNK), jnp.float32),
            jax.ShapeDtypeStruct((N, D), jnp.float32),
        ],
    )(x, w1, b1r, w2t, b2r)


def _b2r(x4):
    return jnp.broadcast_to(x4.reshape(B, 1, 1), (B, RPB, 1)).reshape(R, 1)


def _r2b(x32, red):
    return red(x32.reshape(B, RPB), axis=1, keepdims=True)


def _select_body(l_ref, kl_ref, mask_ref, gidx_ref, ek_ref, sv_ref, idxs_ref, vals_ref):
    kl = kl_ref[...]
    km = jnp.max(kl)
    kp = jnp.exp(kl - km)
    ar = lax.broadcasted_iota(jnp.int32, (1, MAX_K), 1).astype(jnp.float32) + 1.0
    ek = jnp.sum(kp * ar) / jnp.sum(kp)
    ek_ref[...] = ek.reshape(1, 1)
    ki = lax.broadcasted_iota(jnp.int32, (1, MAX_K), 1)
    ksel = jnp.min(jnp.where(kl >= km, ki, MAX_K)) + 1

    v = l_ref[...].reshape(R, C)
    rm = _b2r(_r2b(jnp.max(v, axis=1, keepdims=True), jnp.max))
    e = jnp.exp(v - rm)
    s = e / _b2r(_r2b(jnp.sum(e, axis=1, keepdims=True), jnp.sum))

    rowi = lax.broadcasted_iota(jnp.int32, (R, C), 0)
    lane = lax.broadcasted_iota(jnp.int32, (R, C), 1)
    fib = (rowi & (RPB - 1)) * C + lane
    i64 = lax.broadcasted_iota(jnp.int32, (B, MAX_K), 1)

    sv_ref[...] = s
    idxs_ref[...] = jnp.zeros((B, MAX_K), jnp.int32)
    vals_ref[...] = jnp.zeros((B, MAX_K), jnp.float32)

    def body(j, carry):
        sv = sv_ref[...]
        m4 = _r2b(jnp.max(sv, axis=1, keepdims=True), jnp.max)
        mb = _b2r(m4)
        cand = jnp.where(sv >= mb, fib, -1)
        idx4 = _r2b(jnp.max(cand, axis=1, keepdims=True), jnp.max)
        hit = fib == _b2r(idx4)
        sv_ref[...] = jnp.where(hit, -1.0, sv)
        idxs_ref[...] = jnp.where(i64 == j, idx4, idxs_ref[...])
        vals_ref[...] = jnp.where(i64 == j, m4, vals_ref[...])
        return carry

    lax.fori_loop(0, MAX_K, body, 0)
    idxs = idxs_ref[...]
    vals = vals_ref[...]

    last = i64 == (ksel - 1)
    t4 = jnp.max(jnp.where(last, vals, -1.0), axis=1, keepdims=True)
    il4 = jnp.max(jnp.where(last, idxs, -1), axis=1, keepdims=True)
    tb = _b2r(t4)
    ilb = _b2r(il4)
    selected = jnp.logical_or(s > tb, jnp.logical_and(s >= tb, fib >= ilb))
    mask_ref[...] = jnp.where(selected, (1.0 - s) + s, 0.0).reshape(B, S)

    g = idxs + lax.broadcasted_iota(jnp.int32, (B, MAX_K), 0) * S
    g0 = lax.slice(g, (0, 0), (B, 1))
    gidx_ref[...] = jnp.where(i64 < ksel, g, g0)


def _select(logits3, klr):
    return pl.pallas_call(
        _select_body,
        out_shape=[
            jax.ShapeDtypeStruct((B, S), jnp.float32),
            jax.ShapeDtypeStruct((B, MAX_K), jnp.int32),
            jax.ShapeDtypeStruct((1, 1), jnp.float32),
        ],
        scratch_shapes=[
            pltpu.VMEM((R, C), jnp.float32),
            pltpu.VMEM((B, MAX_K), jnp.int32),
            pltpu.VMEM((B, MAX_K), jnp.float32),
        ],
    )(logits3, klr)


def _sc_scatter_body(x_hbm, gidx_hbm, o_ref, idx_v, rows_v, sem):
    c = lax.axis_index("c")
    sid = lax.axis_index("s")

    @pl.when(sid < 8)
    def _():
        base = (c * 8 + sid) * 16
        pltpu.sync_copy(gidx_hbm.at[pl.ds(base, 16)], idx_v)
        pltpu.async_copy(x_hbm.at[idx_v], rows_v, sem).wait()
        pltpu.async_copy(rows_v, o_ref.at[idx_v], sem).wait()


def _sc_scatter(x, gidx, o_ref):
    mesh = plsc.VectorSubcoreMesh(
        core_axis_name="c", subcore_axis_name="s", num_cores=2, num_subcores=16
    )
    return pl.kernel(
        _sc_scatter_body,
        out_type=(),
        mesh=mesh,
        scratch_types=[
            pltpu.VMEM((16,), jnp.int32),
            pltpu.VMEM((16, D), jnp.float32),
            pltpu.SemaphoreType.DMA,
        ],
    )(x, gidx, o_ref)


def kernel(token_embeddings, W1, b1, W2, b2, k_logits):
    x = token_embeddings.reshape(N, D)
    b1r = b1.reshape(1, HIDDEN)
    w2t = W2.reshape(1, HIDDEN)
    b2r = b2.reshape(1, 1)
    klr = k_logits.reshape(1, MAX_K)

    logits3, zeros = _mlp_zero(x, W1, b1r, w2t, b2r)
    mask, gidx4, ek = _select(logits3, klr)

    o_ref = jax.new_ref(zeros)
    _sc_scatter(x, gidx4.reshape(NSLOT), o_ref)
    filtered = o_ref[...].reshape(B, S, D)
    return filtered, mask, ek.reshape(())


# speedup vs baseline: 1.1416x; 1.1416x over previous
import jax
import jax.numpy as jnp
from jax import lax
from jax.experimental import pallas as pl
from jax.experimental.pallas import tpu as pltpu
from jax.experimental.pallas import tpu_sc as plsc

B, S, D = 4, 8192, 768
HIDDEN = 64
MAX_K = 64
N = B * S
CHUNK = 2048
NSTEP = N // CHUNK
NSLOT = B * MAX_K
G, C = 8, 1024


def _fused_body(x_ref, w1_ref, b1_ref, w2t_ref, b2_ref, kl_ref,
                mask_ref, gidx_ref, ek_ref, o_ref,
                lg_ref, s_ref, sv_ref, idxs_ref, vals_ref):
    i = pl.program_id(0)
    x = x_ref[...]
    h = lax.dot_general(x, w1_ref[...], (((1,), (0,)), ((), ())))
    h = jnp.maximum(h + b1_ref[...], 0.0)
    lg = lax.dot_general(w2t_ref[...], h, (((1,), (1,)), ((), ())))
    lg = lg + b2_ref[...]
    lg_ref[pl.ds(i, 1), :, :] = lg.reshape(1, 1, CHUNK)
    o_ref[...] = jnp.zeros((CHUNK, D), jnp.float32)

    @pl.when(i == NSTEP - 1)
    def _():
        kl = kl_ref[...]
        km = jnp.max(kl)
        kp = jnp.exp(kl - km)
        ar = lax.broadcasted_iota(jnp.int32, (1, MAX_K), 1).astype(jnp.float32)
        ek = jnp.sum(kp * (ar + 1.0)) / jnp.sum(kp)
        ek_ref[...] = ek.reshape(1, 1)
        ki = lax.broadcasted_iota(jnp.int32, (1, MAX_K), 1)
        ksel = jnp.min(jnp.where(kl >= km, ki, MAX_K)) + 1

        v = lg_ref[...].reshape(B, G, C)
        rm = jnp.max(v, axis=(1, 2), keepdims=True)
        e = jnp.exp(v - rm)
        s = e / jnp.sum(e, axis=(1, 2), keepdims=True)
        s_ref[...] = s
        sv_ref[...] = s

        fib = (lax.broadcasted_iota(jnp.int32, (B, G, C), 1) * C
               + lax.broadcasted_iota(jnp.int32, (B, G, C), 2))
        i64 = lax.broadcasted_iota(jnp.int32, (B, MAX_K), 1)

        idxs_ref[...] = jnp.zeros((B, MAX_K), jnp.int32)
        vals_ref[...] = jnp.zeros((B, MAX_K), jnp.float32)

        def body(j, carry):
            sv = sv_ref[...]
            m = jnp.max(sv, axis=(1, 2), keepdims=True)
            cand = jnp.where(sv >= m, fib, -1)
            idx = jnp.max(cand, axis=(1, 2), keepdims=True)
            hit = fib == idx
            sv_ref[...] = jnp.where(hit, -1.0, sv)
            idxs_ref[...] = jnp.where(i64 == j, idx.reshape(B, 1), idxs_ref[...])
            vals_ref[...] = jnp.where(i64 == j, m.reshape(B, 1), vals_ref[...])
            return carry

        lax.fori_loop(0, MAX_K, body, 0)
        idxs = idxs_ref[...]
        vals = vals_ref[...]

        last = i64 == (ksel - 1)
        t = jnp.max(jnp.where(last, vals, -1.0), axis=1, keepdims=True)
        il = jnp.max(jnp.where(last, idxs, -1), axis=1, keepdims=True)
        tb = t.reshape(B, 1, 1)
        ilb = il.reshape(B, 1, 1)
        sfin = s_ref[...]
        selected = jnp.logical_or(
            sfin > tb, jnp.logical_and(sfin >= tb, fib >= ilb))
        mask_ref[...] = jnp.where(selected, (1.0 - sfin) + sfin, 0.0).reshape(B, S)

        g = idxs + lax.broadcasted_iota(jnp.int32, (B, MAX_K), 0) * S
        g0 = lax.slice(g, (0, 0), (B, 1))
        gidx_ref[...] = jnp.where(i64 < ksel, g, g0)


def _fused(x, w1, b1r, w2t, b2r, klr):
    return pl.pallas_call(
        _fused_body,
        grid=(NSTEP,),
        in_specs=[
            pl.BlockSpec((CHUNK, D), lambda i: (i, 0)),
            pl.BlockSpec((D, HIDDEN), lambda i: (0, 0)),
            pl.BlockSpec((1, HIDDEN), lambda i: (0, 0)),
            pl.BlockSpec((1, HIDDEN), lambda i: (0, 0)),
            pl.BlockSpec((1, 1), lambda i: (0, 0)),
            pl.BlockSpec((1, MAX_K), lambda i: (0, 0)),
        ],
        out_specs=[
            pl.BlockSpec((B, S), lambda i: (0, 0)),
            pl.BlockSpec((B, MAX_K), lambda i: (0, 0)),
            pl.BlockSpec((1, 1), lambda i: (0, 0)),
            pl.BlockSpec((CHUNK, D), lambda i: (i, 0)),
        ],
        out_shape=[
            jax.ShapeDtypeStruct((B, S), jnp.float32),
            jax.ShapeDtypeStruct((B, MAX_K), jnp.int32),
            jax.ShapeDtypeStruct((1, 1), jnp.float32),
            jax.ShapeDtypeStruct((N, D), jnp.float32),
        ],
        scratch_shapes=[
            pltpu.VMEM((NSTEP, 1, CHUNK), jnp.float32),
            pltpu.VMEM((B, G, C), jnp.float32),
            pltpu.VMEM((B, G, C), jnp.float32),
            pltpu.VMEM((B, MAX_K), jnp.int32),
            pltpu.VMEM((B, MAX_K), jnp.float32),
        ],
    )(x, w1, b1r, w2t, b2r, klr)


def _sc_scatter_body(x_hbm, gidx_hbm, o_ref, idx_v, rows_v, sem):
    c = lax.axis_index("c")
    sid = lax.axis_index("s")

    @pl.when(sid < 8)
    def _():
        base = (c * 8 + sid) * 16
        pltpu.sync_copy(gidx_hbm.at[pl.ds(base, 16)], idx_v)
        pltpu.async_copy(x_hbm.at[idx_v], rows_v, sem).wait()
        pltpu.async_copy(rows_v, o_ref.at[idx_v], sem).wait()


def _sc_scatter(x, gidx, o_ref):
    mesh = plsc.VectorSubcoreMesh(
        core_axis_name="c", subcore_axis_name="s", num_cores=2, num_subcores=16
    )
    return pl.kernel(
        _sc_scatter_body,
        out_type=(),
        mesh=mesh,
        scratch_types=[
            pltpu.VMEM((16,), jnp.int32),
            pltpu.VMEM((16, D), jnp.float32),
            pltpu.SemaphoreType.DMA,
        ],
    )(x, gidx, o_ref)


def kernel(token_embeddings, W1, b1, W2, b2, k_logits):
    x = token_embeddings.reshape(N, D)
    b1r = b1.reshape(1, HIDDEN)
    w2t = W2.reshape(1, HIDDEN)
    b2r = b2.reshape(1, 1)
    klr = k_logits.reshape(1, MAX_K)

    mask, gidx4, ek, zeros = _fused(x, W1, b1r, w2t, b2r, klr)

    o_ref = jax.new_ref(zeros)
    _sc_scatter(x, gidx4.reshape(NSLOT), o_ref)
    filtered = o_ref[...].reshape(B, S, D)
    return filtered, mask, ek.reshape(())


# speedup vs baseline: 1.2711x; 1.1134x over previous
import jax
import jax.numpy as jnp
from jax import lax
from jax.experimental import pallas as pl
from jax.experimental.pallas import tpu as pltpu
from jax.experimental.pallas import tpu_sc as plsc

B, S, D = 4, 8192, 768
HIDDEN = 64
MAX_K = 64
N = B * S
CHUNK = 2048
NSTEP = N // CHUNK
NSLOT = B * MAX_K
G, C = 8, 1024


def _fused_body(x_ref, w1_ref, b1_ref, w2t_ref, b2_ref, kl_ref,
                mask_ref, gidx_ref, ek_ref, o_ref,
                lg_ref, s_ref):
    i = pl.program_id(0)
    x = x_ref[...]
    h = lax.dot_general(x, w1_ref[...], (((1,), (0,)), ((), ())))
    h = jnp.maximum(h + b1_ref[...], 0.0)
    lg = lax.dot_general(w2t_ref[...], h, (((1,), (1,)), ((), ())))
    lg = lg + b2_ref[...]
    lg_ref[pl.ds(i, 1), :, :] = lg.reshape(1, 1, CHUNK)
    o_ref[...] = jnp.zeros((CHUNK, D), jnp.float32)

    @pl.when(i == NSTEP - 1)
    def _():
        kl = kl_ref[...]
        km = jnp.max(kl)
        kp = jnp.exp(kl - km)
        ar = lax.broadcasted_iota(jnp.int32, (1, MAX_K), 1).astype(jnp.float32)
        ek = jnp.sum(kp * (ar + 1.0)) / jnp.sum(kp)
        ek_ref[...] = ek.reshape(1, 1)
        ki = lax.broadcasted_iota(jnp.int32, (1, MAX_K), 1)
        ksel = jnp.min(jnp.where(kl >= km, ki, MAX_K)) + 1

        v = lg_ref[...].reshape(B, G, C)
        rm = jnp.max(v, axis=(1, 2), keepdims=True)
        e = jnp.exp(v - rm)
        s = e / jnp.sum(e, axis=(1, 2), keepdims=True)
        s_ref[...] = s

        fib = (lax.broadcasted_iota(jnp.int32, (B, G, C), 1) * C
               + lax.broadcasted_iota(jnp.int32, (B, G, C), 2))
        i64 = lax.broadcasted_iota(jnp.int32, (B, MAX_K), 1)

        sb = lax.bitcast_convert_type(s, jnp.int32)

        def count_gt(t):
            return jnp.sum((sb > t).astype(jnp.int32), axis=(1, 2), keepdims=True)

        def vbody(_, lohi):
            lo, hi = lohi
            mid = (lo + hi) >> 1
            c = count_gt(mid)
            return (jnp.where(c >= ksel, mid + 1, lo),
                    jnp.where(c >= ksel, hi, mid))

        z = jnp.zeros((B, 1, 1), jnp.int32)
        tb, _ = lax.fori_loop(0, 31, vbody, (z, z + (1 << 30)))

        ties = sb == tb
        cgt = count_gt(tb)
        extra = ksel - cgt

        def ibody(_, lohi):
            lo, hi = lohi
            mid = (lo + hi + 1) >> 1
            ec = jnp.sum(jnp.logical_and(ties, fib >= mid).astype(jnp.int32),
                         axis=(1, 2), keepdims=True)
            return (jnp.where(ec >= extra, mid, lo),
                    jnp.where(ec >= extra, hi, mid - 1))

        il, _ = lax.fori_loop(0, 13, ibody, (z, z + (S - 1)))

        selected = jnp.logical_or(sb > tb, jnp.logical_and(ties, fib >= il))
        sfin = s_ref[...]
        mask_ref[...] = jnp.where(selected, (1.0 - sfin) + sfin, 0.0).reshape(B, S)

        sel01 = selected.astype(jnp.int32)
        rs = jnp.sum(sel01, axis=2, keepdims=True)
        csr = rs
        k = 1
        while k < G:
            csr = csr + jnp.concatenate(
                [jnp.zeros((B, k, 1), jnp.int32), csr[:, :-k, :]], axis=1)
            k *= 2
        rofs = csr - rs
        csl = sel01
        k = 1
        while k < C:
            csl = csl + jnp.concatenate(
                [jnp.zeros((B, G, k), jnp.int32), csl[:, :, :-k]], axis=2)
            k *= 2
        slot = csl - 1 + rofs
        slotm = jnp.where(selected, slot, -7)

        idxs = jnp.zeros((B, MAX_K), jnp.int32)
        for j in range(MAX_K):
            mj = jnp.max(jnp.where(slotm == j, fib, -1), axis=(1, 2), keepdims=True)
            idxs = jnp.where(i64 == j, mj.reshape(B, 1), idxs)

        g0 = lax.slice(idxs, (0, 0), (B, 1))
        idxs = jnp.where(idxs < 0, g0, idxs)
        gidx_ref[...] = idxs + lax.broadcasted_iota(jnp.int32, (B, MAX_K), 0) * S


def _fused(x, w1, b1r, w2t, b2r, klr):
    return pl.pallas_call(
        _fused_body,
        grid=(NSTEP,),
        in_specs=[
            pl.BlockSpec((CHUNK, D), lambda i: (i, 0)),
            pl.BlockSpec((D, HIDDEN), lambda i: (0, 0)),
            pl.BlockSpec((1, HIDDEN), lambda i: (0, 0)),
            pl.BlockSpec((1, HIDDEN), lambda i: (0, 0)),
            pl.BlockSpec((1, 1), lambda i: (0, 0)),
            pl.BlockSpec((1, MAX_K), lambda i: (0, 0)),
        ],
        out_specs=[
            pl.BlockSpec((B, S), lambda i: (0, 0)),
            pl.BlockSpec((B, MAX_K), lambda i: (0, 0)),
            pl.BlockSpec((1, 1), lambda i: (0, 0)),
            pl.BlockSpec((CHUNK, D), lambda i: (i, 0)),
        ],
        out_shape=[
            jax.ShapeDtypeStruct((B, S), jnp.float32),
            jax.ShapeDtypeStruct((B, MAX_K), jnp.int32),
            jax.ShapeDtypeStruct((1, 1), jnp.float32),
            jax.ShapeDtypeStruct((N, D), jnp.float32),
        ],
        scratch_shapes=[
            pltpu.VMEM((NSTEP, 1, CHUNK), jnp.float32),
            pltpu.VMEM((B, G, C), jnp.float32),
        ],
    )(x, w1, b1r, w2t, b2r, klr)


def _sc_scatter_body(x_hbm, gidx_hbm, o_ref, idx_v, rows_v, sem):
    c = lax.axis_index("c")
    sid = lax.axis_index("s")

    @pl.when(sid < 8)
    def _():
        base = (c * 8 + sid) * 16
        pltpu.sync_copy(gidx_hbm.at[pl.ds(base, 16)], idx_v)
        pltpu.async_copy(x_hbm.at[idx_v], rows_v, sem).wait()
        pltpu.async_copy(rows_v, o_ref.at[idx_v], sem).wait()


def _sc_scatter(x, gidx, o_ref):
    mesh = plsc.VectorSubcoreMesh(
        core_axis_name="c", subcore_axis_name="s", num_cores=2, num_subcores=16
    )
    return pl.kernel(
        _sc_scatter_body,
        out_type=(),
        mesh=mesh,
        scratch_types=[
            pltpu.VMEM((16,), jnp.int32),
            pltpu.VMEM((16, D), jnp.float32),
            pltpu.SemaphoreType.DMA,
        ],
    )(x, gidx, o_ref)


def kernel(token_embeddings, W1, b1, W2, b2, k_logits):
    x = token_embeddings.reshape(N, D)
    b1r = b1.reshape(1, HIDDEN)
    w2t = W2.reshape(1, HIDDEN)
    b2r = b2.reshape(1, 1)
    klr = k_logits.reshape(1, MAX_K)

    mask, gidx4, ek, zeros = _fused(x, W1, b1r, w2t, b2r, klr)

    o_ref = jax.new_ref(zeros)
    _sc_scatter(x, gidx4.reshape(NSLOT), o_ref)
    filtered = o_ref[...].reshape(B, S, D)
    return filtered, mask, ek.reshape(())
